# Initial kernel scaffold; baseline (speedup 1.0000x reference)
import functools

import jax
import jax.numpy as jnp
from jax.experimental import pallas as pl
from jax.experimental.pallas import tpu as pltpu
from jax.experimental.pallas import tpu_sc as plsc

_N = 10000
_E = 160000
_D = 128
_R = 4

_NC = 2
_NS = 16
_B = 128
_NB_TILE = 40
_E_PAD = _NC * _NS * _NB_TILE * _B
_ACC_ROWS = 10240
_ROWS_PER_SUB = _ACC_ROWS // _NS
_OUT_PER_SUB = _N // _NS


def _mm_body(x_ref, w_ref, o_ref):
    o_ref[...] = jnp.dot(x_ref[...], w_ref[0],
                         preferred_element_type=jnp.float32)


def _matmul_all(x, wcat):
    nbx = 5
    bn = _N // nbx
    return pl.pallas_call(
        _mm_body,
        grid=(_R + 1, nbx),
        in_specs=[
            pl.BlockSpec((bn, _D), lambda r, i: (i, 0)),
            pl.BlockSpec((1, _D, _D), lambda r, i: (r, 0, 0)),
        ],
        out_specs=pl.BlockSpec((bn, _D), lambda r, i: (r * nbx + i, 0)),
        out_shape=jax.ShapeDtypeStruct(((_R + 1) * _N, _D), jnp.float32),
    )(x, wcat)


def _gidx_body(src_ref, et_ref, o_ref):
    o_ref[...] = (et_ref[...] + 1) * _N + src_ref[...]


def _make_gidx(src2d, et2d):
    nrows = _E_PAD // _B
    return pl.pallas_call(
        _gidx_body,
        grid=(8,),
        in_specs=[
            pl.BlockSpec((nrows // 8, _B), lambda i: (i, 0)),
            pl.BlockSpec((nrows // 8, _B), lambda i: (i, 0)),
        ],
        out_specs=pl.BlockSpec((nrows // 8, _B), lambda i: (i, 0)),
        out_shape=jax.ShapeDtypeStruct((nrows, _B), jnp.int32),
    )(src2d, et2d)


def _combine_body(y_ref, a0_ref, a1_ref, x_ref, b_ref, o_ref, *, residual):
    o = y_ref[...] + a0_ref[...] + a1_ref[...] + b_ref[...]
    if residual:
        o = o + x_ref[...]
    o_ref[...] = jnp.maximum(o, 0.0)


def _combine(yall, agg0, agg1, x, b2d, residual):
    nbx = 5
    bn = _N // nbx
    return pl.pallas_call(
        functools.partial(_combine_body, residual=residual),
        grid=(nbx,),
        in_specs=[
            pl.BlockSpec((bn, _D), lambda i: (i, 0)),
            pl.BlockSpec((bn, _D), lambda i: (i, 0)),
            pl.BlockSpec((bn, _D), lambda i: (i, 0)),
            pl.BlockSpec((bn, _D), lambda i: (i, 0)),
            pl.BlockSpec((1, _D), lambda i: (0, 0)),
        ],
        out_specs=pl.BlockSpec((bn, _D), lambda i: (i, 0)),
        out_shape=jax.ShapeDtypeStruct((_N, _D), jnp.float32),
    )(yall, agg0, agg1, x, b2d)


def _sc_body(yall, gidx_hbm, dst_hbm, out0, out1,
             acc, gidx_v, dst_v, rows_v, sem):
    c = jax.lax.axis_index("c")
    s = jax.lax.axis_index("s")
    wid = c * _NS + s

    def _zero_row(i, _):
        for k in range(_D // 16):
            rows_v[i, pl.ds(k * 16, 16)] = jnp.zeros((16,), jnp.float32)
        return 0
    jax.lax.fori_loop(0, _B, _zero_row, 0)
    base = s * _ROWS_PER_SUB
    for j in range(_ROWS_PER_SUB // _B):
        pltpu.sync_copy(rows_v, acc.at[pl.ds(base + j * _B, _B)])
    plsc.subcore_barrier()

    pltpu.sync_copy(gidx_hbm.at[pl.ds(wid * _NB_TILE, _NB_TILE)], gidx_v)
    pltpu.sync_copy(dst_hbm.at[pl.ds(wid * _NB_TILE, _NB_TILE)], dst_v)

    def _batch(b, _):
        pltpu.async_copy(yall.at[gidx_v.at[b]], rows_v, sem).wait()
        pltpu.sync_copy(rows_v, acc.at[dst_v.at[b]], add=True)
        return 0
    jax.lax.fori_loop(0, _NB_TILE, _batch, 0)
    plsc.subcore_barrier()

    ob = s * _OUT_PER_SUB
    for j in range(5):
        r0 = ob + j * 125

        @pl.when(c == 0)
        def _():
            pltpu.sync_copy(acc.at[pl.ds(r0, 125)], out0.at[pl.ds(r0, 125)])

        @pl.when(c == 1)
        def _():
            pltpu.sync_copy(acc.at[pl.ds(r0, 125)], out1.at[pl.ds(r0, 125)])


def _sc_aggregate(yall, gidx2d, dst2d):
    mesh = plsc.VectorSubcoreMesh(core_axis_name="c", subcore_axis_name="s")
    f = pl.kernel(
        _sc_body,
        out_type=[jax.ShapeDtypeStruct((_N, _D), jnp.float32),
                  jax.ShapeDtypeStruct((_N, _D), jnp.float32)],
        mesh=mesh,
        scratch_types=[
            pltpu.VMEM_SHARED((_ACC_ROWS, _D), jnp.float32),
            pltpu.VMEM((_NB_TILE, _B), jnp.int32),
            pltpu.VMEM((_NB_TILE, _B), jnp.int32),
            pltpu.VMEM((_B, _D), jnp.float32),
            pltpu.SemaphoreType.DMA,
        ],
    )
    return f(yall, gidx2d, dst2d)


@jax.jit
def kernel(x, edge_index, edge_type, W1, S1, b1, W2, S2, b2):
    src = edge_index[0].astype(jnp.int32)
    dst = edge_index[1].astype(jnp.int32)
    et = edge_type.astype(jnp.int32)

    pad = _E_PAD - _E
    src2d = jnp.concatenate(
        [src, jnp.zeros((pad,), jnp.int32)]).reshape(_E_PAD // _B, _B)
    et2d = jnp.concatenate(
        [et, jnp.zeros((pad,), jnp.int32)]).reshape(_E_PAD // _B, _B)
    dst2d = jnp.concatenate(
        [dst, jnp.full((pad,), _N, jnp.int32)]).reshape(_E_PAD // _B, _B)
    gidx2d = _make_gidx(src2d, et2d)

    wcat1 = jnp.concatenate([S1[None], W1], axis=0)
    wcat2 = jnp.concatenate([S2[None], W2], axis=0)
    b1r = b1.reshape(1, _D)
    b2r = b2.reshape(1, _D)

    yall1 = _matmul_all(x, wcat1)
    agg0, agg1 = _sc_aggregate(yall1, gidx2d, dst2d)
    h = _combine(yall1, agg0, agg1, x, b1r, residual=True)

    yall2 = _matmul_all(h, wcat2)
    agg0b, agg1b = _sc_aggregate(yall2, gidx2d, dst2d)
    out = _combine(yall2, agg0b, agg1b, h, b2r, residual=False)
    return out


# speedup vs baseline: 3.6856x; 3.6856x over previous
import functools

import jax
import jax.numpy as jnp
from jax.experimental import pallas as pl
from jax.experimental.pallas import tpu as pltpu
from jax.experimental.pallas import tpu_sc as plsc

_N = 10000
_E = 160000
_D = 128
_R = 4

_NC = 2
_NS = 16
_B = 128
_NB_TILE = 40
_E_PAD = _NC * _NS * _NB_TILE * _B
_ACC_ROWS = 10240
_ROWS_PER_SUB = _ACC_ROWS // _NS
_OUT_PER_SUB = _N // _NS


def _mm_body(x_ref, w_ref, o_ref):
    o_ref[...] = jnp.dot(x_ref[...], w_ref[0],
                         preferred_element_type=jnp.float32)


def _matmul_all(x, wcat):
    nbx = 5
    bn = _N // nbx
    return pl.pallas_call(
        _mm_body,
        grid=(_R + 1, nbx),
        in_specs=[
            pl.BlockSpec((bn, _D), lambda r, i: (i, 0)),
            pl.BlockSpec((1, _D, _D), lambda r, i: (r, 0, 0)),
        ],
        out_specs=pl.BlockSpec((bn, _D), lambda r, i: (r * nbx + i, 0)),
        out_shape=jax.ShapeDtypeStruct(((_R + 1) * _N, _D), jnp.float32),
    )(x, wcat)


def _gidx_body(src_ref, et_ref, o_ref):
    o_ref[...] = (et_ref[...] + 1) * _N + src_ref[...]


def _make_gidx(src2d, et2d):
    nrows = _E_PAD // _B
    return pl.pallas_call(
        _gidx_body,
        grid=(8,),
        in_specs=[
            pl.BlockSpec((nrows // 8, _B), lambda i: (i, 0)),
            pl.BlockSpec((nrows // 8, _B), lambda i: (i, 0)),
        ],
        out_specs=pl.BlockSpec((nrows // 8, _B), lambda i: (i, 0)),
        out_shape=jax.ShapeDtypeStruct((nrows, _B), jnp.int32),
    )(src2d, et2d)


def _combine_body(y_ref, a0_ref, a1_ref, x_ref, b_ref, o_ref, *, residual):
    o = y_ref[...] + a0_ref[...] + a1_ref[...] + b_ref[...]
    if residual:
        o = o + x_ref[...]
    o_ref[...] = jnp.maximum(o, 0.0)


def _combine(yall, agg0, agg1, x, b2d, residual):
    nbx = 5
    bn = _N // nbx
    return pl.pallas_call(
        functools.partial(_combine_body, residual=residual),
        grid=(nbx,),
        in_specs=[
            pl.BlockSpec((bn, _D), lambda i: (i, 0)),
            pl.BlockSpec((bn, _D), lambda i: (i, 0)),
            pl.BlockSpec((bn, _D), lambda i: (i, 0)),
            pl.BlockSpec((bn, _D), lambda i: (i, 0)),
            pl.BlockSpec((1, _D), lambda i: (0, 0)),
        ],
        out_specs=pl.BlockSpec((bn, _D), lambda i: (i, 0)),
        out_shape=jax.ShapeDtypeStruct((_N, _D), jnp.float32),
    )(yall, agg0, agg1, x, b2d)


def _sc_body(yall, gidx_hbm, dst_hbm, out0, out1,
             acc, gidx_v, dst_v, rows_v, sem):
    c = jax.lax.axis_index("c")
    s = jax.lax.axis_index("s")
    wid = c * _NS + s

    def _zero_row(i, _):
        for k in range(_D // 16):
            rows_v[i, pl.ds(k * 16, 16)] = jnp.zeros((16,), jnp.float32)
        return 0
    jax.lax.fori_loop(0, _B, _zero_row, 0)
    base = s * _ROWS_PER_SUB
    for j in range(_ROWS_PER_SUB // _B):
        pltpu.sync_copy(rows_v, acc.at[pl.ds(base + j * _B, _B)])
    plsc.subcore_barrier()

    pltpu.sync_copy(gidx_hbm.at[pl.ds(wid * _NB_TILE, _NB_TILE)], gidx_v)
    pltpu.sync_copy(dst_hbm.at[pl.ds(wid * _NB_TILE, _NB_TILE)], dst_v)

    def _batch(b, _):
        pltpu.async_copy(yall.at[gidx_v.at[b]], rows_v, sem).wait()
        pltpu.sync_copy(rows_v, acc.at[dst_v.at[b]], add=True)
        return 0
    jax.lax.fori_loop(0, _NB_TILE, _batch, 0)
    plsc.subcore_barrier()

    for cc, out_ref in ((0, out0), (1, out1)):
        @pl.when(c == cc)
        def _():
            @pl.when(s < _NS - 1)
            def _():
                r0 = pl.multiple_of(s * 624, 8)
                pltpu.sync_copy(acc.at[pl.ds(r0, 624)],
                                out_ref.at[pl.ds(r0, 624)])

            @pl.when(s == _NS - 1)
            def _():
                pltpu.sync_copy(acc.at[pl.ds(9360, 640)],
                                out_ref.at[pl.ds(9360, 640)])


def _sc_aggregate(yall, gidx2d, dst2d):
    mesh = plsc.VectorSubcoreMesh(core_axis_name="c", subcore_axis_name="s")
    f = pl.kernel(
        _sc_body,
        out_type=[jax.ShapeDtypeStruct((_N, _D), jnp.float32),
                  jax.ShapeDtypeStruct((_N, _D), jnp.float32)],
        mesh=mesh,
        scratch_types=[
            pltpu.VMEM_SHARED((_ACC_ROWS, _D), jnp.float32),
            pltpu.VMEM((_NB_TILE, _B), jnp.int32),
            pltpu.VMEM((_NB_TILE, _B), jnp.int32),
            pltpu.VMEM((_B, _D), jnp.float32),
            pltpu.SemaphoreType.DMA,
        ],
    )
    return f(yall, gidx2d, dst2d)


@jax.jit
def kernel(x, edge_index, edge_type, W1, S1, b1, W2, S2, b2):
    src = edge_index[0].astype(jnp.int32)
    dst = edge_index[1].astype(jnp.int32)
    et = edge_type.astype(jnp.int32)

    pad = _E_PAD - _E
    src2d = jnp.concatenate(
        [src, jnp.zeros((pad,), jnp.int32)]).reshape(_E_PAD // _B, _B)
    et2d = jnp.concatenate(
        [et, jnp.zeros((pad,), jnp.int32)]).reshape(_E_PAD // _B, _B)
    dst2d = jnp.concatenate(
        [dst, jnp.full((pad,), _N, jnp.int32)]).reshape(_E_PAD // _B, _B)
    gidx2d = _make_gidx(src2d, et2d)

    wcat1 = jnp.concatenate([S1[None], W1], axis=0)
    wcat2 = jnp.concatenate([S2[None], W2], axis=0)
    b1r = b1.reshape(1, _D)
    b2r = b2.reshape(1, _D)

    yall1 = _matmul_all(x, wcat1)
    agg0, agg1 = _sc_aggregate(yall1, gidx2d, dst2d)
    h = _combine(yall1, agg0, agg1, x, b1r, residual=True)

    yall2 = _matmul_all(h, wcat2)
    agg0b, agg1b = _sc_aggregate(yall2, gidx2d, dst2d)
    out = _combine(yall2, agg0b, agg1b, h, b2r, residual=False)
    return out


# speedup vs baseline: 4.0404x; 1.0962x over previous
import functools

import jax
import jax.numpy as jnp
from jax.experimental import pallas as pl
from jax.experimental.pallas import tpu as pltpu
from jax.experimental.pallas import tpu_sc as plsc

_N = 10000
_E = 160000
_D = 128
_R = 4

_NC = 2
_NS = 16
_B = 128
_NB_TILE = 40
_E_PAD = _NC * _NS * _NB_TILE * _B
_ACC_ROWS = 10240
_ROWS_PER_SUB = _ACC_ROWS // _NS
_OUT_PER_SUB = _N // _NS


def _mm_body(x_ref, w_ref, o_ref):
    o_ref[...] = jnp.dot(x_ref[...], w_ref[0],
                         preferred_element_type=jnp.float32)


def _matmul_all(x, wcat):
    nbx = 5
    bn = _N // nbx
    return pl.pallas_call(
        _mm_body,
        grid=(_R + 1, nbx),
        in_specs=[
            pl.BlockSpec((bn, _D), lambda r, i: (i, 0)),
            pl.BlockSpec((1, _D, _D), lambda r, i: (r, 0, 0)),
        ],
        out_specs=pl.BlockSpec((bn, _D), lambda r, i: (r * nbx + i, 0)),
        out_shape=jax.ShapeDtypeStruct(((_R + 1) * _N, _D), jnp.float32),
    )(x, wcat)


def _gidx_body(src_ref, et_ref, o_ref):
    o_ref[...] = (et_ref[...] + 1) * _N + src_ref[...]


def _make_gidx(src2d, et2d):
    nrows = _E_PAD // _B
    return pl.pallas_call(
        _gidx_body,
        grid=(8,),
        in_specs=[
            pl.BlockSpec((nrows // 8, _B), lambda i: (i, 0)),
            pl.BlockSpec((nrows // 8, _B), lambda i: (i, 0)),
        ],
        out_specs=pl.BlockSpec((nrows // 8, _B), lambda i: (i, 0)),
        out_shape=jax.ShapeDtypeStruct((nrows, _B), jnp.int32),
    )(src2d, et2d)


def _combine_body(y_ref, a0_ref, a1_ref, x_ref, b_ref, o_ref, *, residual):
    o = y_ref[...] + a0_ref[...] + a1_ref[...] + b_ref[...]
    if residual:
        o = o + x_ref[...]
    o_ref[...] = jnp.maximum(o, 0.0)


def _combine(yall, agg0, agg1, x, b2d, residual):
    nbx = 5
    bn = _N // nbx
    return pl.pallas_call(
        functools.partial(_combine_body, residual=residual),
        grid=(nbx,),
        in_specs=[
            pl.BlockSpec((bn, _D), lambda i: (i, 0)),
            pl.BlockSpec((bn, _D), lambda i: (i, 0)),
            pl.BlockSpec((bn, _D), lambda i: (i, 0)),
            pl.BlockSpec((bn, _D), lambda i: (i, 0)),
            pl.BlockSpec((1, _D), lambda i: (0, 0)),
        ],
        out_specs=pl.BlockSpec((bn, _D), lambda i: (i, 0)),
        out_shape=jax.ShapeDtypeStruct((_N, _D), jnp.float32),
    )(yall, agg0, agg1, x, b2d)


_DEPTH = 2


def _sc_body(yall, gidx_hbm, dst_hbm, out0, out1,
             acc, gidx_v, dst_v, rows_v, *sems):
    c = jax.lax.axis_index("c")
    s = jax.lax.axis_index("s")
    wid = c * _NS + s

    def _zero_row(i, _):
        for k in range(_D // 16):
            rows_v[0, i, pl.ds(k * 16, 16)] = jnp.zeros((16,), jnp.float32)
        return 0
    jax.lax.fori_loop(0, _B, _zero_row, 0)
    base = s * _ROWS_PER_SUB
    for j in range(_ROWS_PER_SUB // _B):
        pltpu.sync_copy(rows_v.at[0], acc.at[pl.ds(base + j * _B, _B)])
    plsc.subcore_barrier()

    pltpu.sync_copy(gidx_hbm.at[pl.ds(wid * _NB_TILE, _NB_TILE)], gidx_v)
    pltpu.sync_copy(dst_hbm.at[pl.ds(wid * _NB_TILE, _NB_TILE)], dst_v)

    def _start(b, k):
        pltpu.async_copy(yall.at[gidx_v.at[b]], rows_v.at[k], sems[k])

    def _wait(b, k):
        pltpu.make_async_copy(yall.at[gidx_v.at[b]], rows_v.at[k],
                              sems[k]).wait()

    for k in range(_DEPTH):
        _start(k, k)

    def _step(i, _):
        for k in range(_DEPTH):
            b = i * _DEPTH + k
            _wait(b, k)
            pltpu.sync_copy(rows_v.at[k], acc.at[dst_v.at[b]], add=True)
            nxt = b + _DEPTH

            @pl.when(nxt < _NB_TILE)
            def _():
                _start(nxt, k)
        return 0
    jax.lax.fori_loop(0, _NB_TILE // _DEPTH, _step, 0)
    plsc.subcore_barrier()

    for cc, out_ref in ((0, out0), (1, out1)):
        @pl.when(c == cc)
        def _():
            @pl.when(s < _NS - 1)
            def _():
                r0 = pl.multiple_of(s * 624, 8)
                pltpu.sync_copy(acc.at[pl.ds(r0, 624)],
                                out_ref.at[pl.ds(r0, 624)])

            @pl.when(s == _NS - 1)
            def _():
                pltpu.sync_copy(acc.at[pl.ds(9360, 640)],
                                out_ref.at[pl.ds(9360, 640)])


def _sc_aggregate(yall, gidx2d, dst2d):
    mesh = plsc.VectorSubcoreMesh(core_axis_name="c", subcore_axis_name="s")
    f = pl.kernel(
        _sc_body,
        out_type=[jax.ShapeDtypeStruct((_N, _D), jnp.float32),
                  jax.ShapeDtypeStruct((_N, _D), jnp.float32)],
        mesh=mesh,
        scratch_types=[
            pltpu.VMEM_SHARED((_ACC_ROWS, _D), jnp.float32),
            pltpu.VMEM((_NB_TILE, _B), jnp.int32),
            pltpu.VMEM((_NB_TILE, _B), jnp.int32),
            pltpu.VMEM((_DEPTH, _B, _D), jnp.float32),
        ] + [pltpu.SemaphoreType.DMA] * _DEPTH,
    )
    return f(yall, gidx2d, dst2d)


@jax.jit
def kernel(x, edge_index, edge_type, W1, S1, b1, W2, S2, b2):
    src = edge_index[0].astype(jnp.int32)
    dst = edge_index[1].astype(jnp.int32)
    et = edge_type.astype(jnp.int32)

    pad = _E_PAD - _E
    src2d = jnp.concatenate(
        [src, jnp.zeros((pad,), jnp.int32)]).reshape(_E_PAD // _B, _B)
    et2d = jnp.concatenate(
        [et, jnp.zeros((pad,), jnp.int32)]).reshape(_E_PAD // _B, _B)
    dst2d = jnp.concatenate(
        [dst, jnp.full((pad,), _N, jnp.int32)]).reshape(_E_PAD // _B, _B)
    gidx2d = _make_gidx(src2d, et2d)

    wcat1 = jnp.concatenate([S1[None], W1], axis=0)
    wcat2 = jnp.concatenate([S2[None], W2], axis=0)
    b1r = b1.reshape(1, _D)
    b2r = b2.reshape(1, _D)

    yall1 = _matmul_all(x, wcat1)
    agg0, agg1 = _sc_aggregate(yall1, gidx2d, dst2d)
    h = _combine(yall1, agg0, agg1, x, b1r, residual=True)

    yall2 = _matmul_all(h, wcat2)
    agg0b, agg1b = _sc_aggregate(yall2, gidx2d, dst2d)
    out = _combine(yall2, agg0b, agg1b, h, b2r, residual=False)
    return out
